# baseline (speedup 1.0000x reference)
import functools

import jax
import jax.numpy as jnp
from jax import lax
from jax.experimental import pallas as pl
from jax.experimental.pallas import tpu as pltpu
from jax.experimental.pallas import tpu_sc as plsc

N = 10000
E = 320000
D = 128
G = 64
O = 16

NC = 2
NS = 16
CH = 128

CPT = 80
E_PAD = NC * NS * CPT * CH
NP = 10240
ROWS_IO = NP // NS
ZCH = NP // NS

_MESH = plsc.VectorSubcoreMesh(core_axis_name="c", subcore_axis_name="s")


@functools.partial(
    pl.kernel,
    out_type=jax.ShapeDtypeStruct((NC * NP,), jnp.float32),
    mesh=_MESH,
    scratch_types=[
        pltpu.VMEM((CPT, CH), jnp.int32),
        pltpu.VMEM((CH,), jnp.float32),
        pltpu.VMEM((ZCH,), jnp.float32),
        pltpu.VMEM_SHARED((NP,), jnp.float32),
    ],
)
def _sc_degree(dst_hbm, out_hbm, dst_v, ones_v, zeros_v, deg_sh):
    c = lax.axis_index("c")
    s = lax.axis_index("s")
    wid = c * NS + s

    for k in range(CH // 16):
        ones_v[pl.ds(k * 16, 16)] = jnp.ones((16,), jnp.float32)

    def zbody(k, _):
        zeros_v[pl.ds(k * 16, 16)] = jnp.zeros((16,), jnp.float32)
        return ()

    lax.fori_loop(0, ZCH // 16, zbody, ())
    pltpu.sync_copy(zeros_v, deg_sh.at[pl.ds(s * ZCH, ZCH)])
    pltpu.sync_copy(dst_hbm.at[pl.ds(wid * CPT, CPT)], dst_v)
    plsc.subcore_barrier()

    def body(i, _):
        pltpu.sync_copy(ones_v, deg_sh.at[dst_v.at[i]], add=True)
        return ()

    lax.fori_loop(0, CPT, body, ())
    plsc.subcore_barrier()
    pltpu.sync_copy(deg_sh.at[pl.ds(s * ZCH, ZCH)],
                    out_hbm.at[pl.ds(c * NP + s * ZCH, ZCH)])


NBUF = 8
CH2 = 32
CPT2 = E_PAD // (NC * NS * CH2)
BLK2 = 32
NQ = BLK2 // NBUF


@functools.partial(
    pl.kernel,
    out_type=jax.ShapeDtypeStruct((NC, NP, D), jnp.float32),
    mesh=_MESH,
    scratch_types=[
        pltpu.VMEM((BLK2, CH2), jnp.int32),
        pltpu.VMEM((BLK2, CH2), jnp.int32),
        [pltpu.VMEM((CH2, D), jnp.float32) for _ in range(NBUF)],
        pltpu.VMEM((CH2,), jnp.int32),
        pltpu.VMEM_SHARED((NP, D), jnp.float32),
        [pltpu.SemaphoreType.DMA for _ in range(NBUF)],
        [pltpu.SemaphoreType.DMA for _ in range(NBUF)],
    ],
)
def _sc_aggregate(g_hbm, src_hbm, dst_hbm, out_hbm,
                  src_v, dst_v, rows, prime_v, acc_sh, sem_g, sem_s):
    c = lax.axis_index("c")
    s = lax.axis_index("s")
    wid = c * NS + s

    pltpu.sync_copy(g_hbm.at[pl.ds(s * ROWS_IO, ROWS_IO)],
                    acc_sh.at[pl.ds(s * ROWS_IO, ROWS_IO)])
    for k in range(CH2 // 16):
        prime_v[pl.ds(k * 16, 16)] = lax.iota(jnp.int32, 16) + (N + 16 * k)
    plsc.subcore_barrier()

    for b in range(NBUF):
        pltpu.async_copy(rows[b], acc_sh.at[prime_v], sem_s[b], add=True)

    def blk(blki, _):
        row0 = wid * CPT2 + blki * BLK2
        pltpu.sync_copy(src_hbm.at[pl.ds(row0, BLK2)], src_v)
        pltpu.sync_copy(dst_hbm.at[pl.ds(row0, BLK2)], dst_v)

        def quad(qi, _):
            gds = []
            for b in range(NBUF):
                pltpu.make_async_copy(rows[b], acc_sh.at[pl.ds(0, CH2)],
                                      sem_s[b]).wait()
                gds.append(pltpu.async_copy(
                    g_hbm.at[src_v.at[qi * NBUF + b]], rows[b], sem_g[b]))
            for b in range(NBUF):
                gds[b].wait()
                pltpu.async_copy(rows[b], acc_sh.at[dst_v.at[qi * NBUF + b]],
                                 sem_s[b], add=True)
            return ()

        lax.fori_loop(0, NQ, quad, ())
        return ()

    lax.fori_loop(0, CPT2 // BLK2, blk, ())
    for b in range(NBUF):
        pltpu.make_async_copy(rows[b], acc_sh.at[pl.ds(0, CH2)],
                              sem_s[b]).wait()
    plsc.subcore_barrier()
    pltpu.sync_copy(acc_sh.at[pl.ds(s * ROWS_IO, ROWS_IO)],
                    out_hbm.at[c, pl.ds(s * ROWS_IO, ROWS_IO)])


def _tc_prep_body(deg_ref, x_ref, w_ref, g_ref, dinv_ref):
    deg = deg_ref[0] + deg_ref[1] + 1.0
    dinv = lax.rsqrt(deg)
    g_ref[:N] = jnp.dot(x_ref[...], w_ref[...],
                        preferred_element_type=jnp.float32) * dinv
    dinv_ref[...] = dinv


def _tc_prep(deg2, x, w1):
    return pl.pallas_call(
        _tc_prep_body,
        out_shape=[jax.ShapeDtypeStruct((NP, D), jnp.float32),
                   jax.ShapeDtypeStruct((N, 1), jnp.float32)],
    )(deg2, x, w1)


def _tc_mid_body(s_ref, g1_ref, dinv_ref, b1_ref, w2_ref, g2_ref):
    dinv = dinv_ref[...]
    pre = (s_ref[0, :N] + s_ref[1, :N] - g1_ref[:N]) * dinv + b1_ref[...]
    h = jnp.tanh(pre)
    g2_ref[:N] = jnp.dot(h, w2_ref[...],
                         preferred_element_type=jnp.float32) * dinv


def _tc_mid(s, g1, dinv, b1, w2):
    return pl.pallas_call(
        _tc_mid_body,
        out_shape=jax.ShapeDtypeStruct((NP, D), jnp.float32),
    )(s, g1, dinv, b1, w2)


def _tc_final_body(t_ref, g2_ref, dinv_ref, b2_ref, batch_ref, wfc_ref,
                   bfc_ref, out_ref):
    pre = (t_ref[0, :N] + t_ref[1, :N] - g2_ref[:N]) * dinv_ref[...] + b2_ref[...]
    h = jnp.tanh(pre)
    ids = lax.broadcasted_iota(jnp.int32, (G, N), 0)
    ind = (ids == batch_ref[...]).astype(jnp.float32)
    sums = jnp.dot(ind, h, preferred_element_type=jnp.float32)
    cnt = jnp.sum(ind, axis=1, keepdims=True)
    pooled = sums / jnp.maximum(cnt, 1.0)
    out_ref[...] = jnp.dot(pooled, wfc_ref[...],
                           preferred_element_type=jnp.float32) + bfc_ref[...]


def _tc_final(t, g2, dinv, b2, batch2d, wfc, bfc):
    return pl.pallas_call(
        _tc_final_body,
        out_shape=jax.ShapeDtypeStruct((G, O), jnp.float32),
    )(t, g2, dinv, b2, batch2d, wfc, bfc)


def kernel(x, edge_index, batch, W1, b1, W2, b2, Wfc, bfc):
    src = edge_index[0]
    dst = edge_index[1]
    pad = E_PAD - E
    pad_idx = jnp.arange(pad, dtype=jnp.int32)
    src_p = jnp.concatenate([src, pad_idx % N])
    dst_p = jnp.concatenate([dst, N + pad_idx % (NP - N)])
    src_a = src_p.reshape(E_PAD // CH2, CH2)
    dst_a = dst_p.reshape(E_PAD // CH2, CH2)

    deg2 = _sc_degree(dst_p.reshape(E_PAD // CH, CH))
    deg2 = deg2.reshape(NC, NP)[:, :N, None]
    g1, dinv = _tc_prep(deg2, x, W1)
    s = _sc_aggregate(g1, src_a, dst_a)
    g2 = _tc_mid(s, g1, dinv, b1.reshape(1, D), W2)
    t = _sc_aggregate(g2, src_a, dst_a)
    out = _tc_final(t, g2, dinv, b2.reshape(1, D), batch.reshape(1, N),
                    Wfc, bfc.reshape(1, O))
    return out


# speedup vs baseline: 1.0222x; 1.0222x over previous
import functools

import jax
import jax.numpy as jnp
from jax import lax
from jax.experimental import pallas as pl
from jax.experimental.pallas import tpu as pltpu
from jax.experimental.pallas import tpu_sc as plsc

N = 10000
E = 320000
D = 128
G = 64
O = 16

NC = 2
NS = 16
CH = 128

CPT = 80
E_PAD = NC * NS * CPT * CH
NP = 10240
ROWS_IO = NP // NS
ZCH = NP // NS

_MESH = plsc.VectorSubcoreMesh(core_axis_name="c", subcore_axis_name="s")


@functools.partial(
    pl.kernel,
    out_type=jax.ShapeDtypeStruct((NC * NP,), jnp.float32),
    mesh=_MESH,
    scratch_types=[
        pltpu.VMEM((CPT, CH), jnp.int32),
        pltpu.VMEM((CH,), jnp.float32),
        pltpu.VMEM((ZCH,), jnp.float32),
        pltpu.VMEM_SHARED((NP,), jnp.float32),
    ],
)
def _sc_degree(dst_hbm, out_hbm, dst_v, ones_v, zeros_v, deg_sh):
    c = lax.axis_index("c")
    s = lax.axis_index("s")
    wid = c * NS + s

    for k in range(CH // 16):
        ones_v[pl.ds(k * 16, 16)] = jnp.ones((16,), jnp.float32)

    def zbody(k, _):
        zeros_v[pl.ds(k * 16, 16)] = jnp.zeros((16,), jnp.float32)
        return ()

    lax.fori_loop(0, ZCH // 16, zbody, ())
    pltpu.sync_copy(zeros_v, deg_sh.at[pl.ds(s * ZCH, ZCH)])
    pltpu.sync_copy(dst_hbm.at[pl.ds(wid * CPT, CPT)], dst_v)
    plsc.subcore_barrier()

    def body(i, _):
        pltpu.sync_copy(ones_v, deg_sh.at[dst_v.at[i]], add=True)
        return ()

    lax.fori_loop(0, CPT, body, ())
    plsc.subcore_barrier()
    pltpu.sync_copy(deg_sh.at[pl.ds(s * ZCH, ZCH)],
                    out_hbm.at[pl.ds(c * NP + s * ZCH, ZCH)])


NBUF = 4
CH2 = 64
CPT2 = E_PAD // (NC * NS * CH2)
BLK2 = 32
NQ = BLK2 // NBUF


@functools.partial(
    pl.kernel,
    out_type=jax.ShapeDtypeStruct((NC, NP, D), jnp.float32),
    mesh=_MESH,
    scratch_types=[
        pltpu.VMEM((BLK2, CH2), jnp.int32),
        pltpu.VMEM((BLK2, CH2), jnp.int32),
        [pltpu.VMEM((CH2, D), jnp.float32) for _ in range(NBUF)],
        pltpu.VMEM((CH2,), jnp.int32),
        pltpu.VMEM_SHARED((NP, D), jnp.float32),
        [pltpu.SemaphoreType.DMA for _ in range(NBUF)],
        [pltpu.SemaphoreType.DMA for _ in range(NBUF)],
    ],
)
def _sc_aggregate(g_hbm, src_hbm, dst_hbm, out_hbm,
                  src_v, dst_v, rows, prime_v, acc_sh, sem_g, sem_s):
    c = lax.axis_index("c")
    s = lax.axis_index("s")
    wid = c * NS + s

    pltpu.sync_copy(g_hbm.at[pl.ds(s * ROWS_IO, ROWS_IO)],
                    acc_sh.at[pl.ds(s * ROWS_IO, ROWS_IO)])
    for k in range(CH2 // 16):
        prime_v[pl.ds(k * 16, 16)] = lax.iota(jnp.int32, 16) + (N + 16 * k)
    plsc.subcore_barrier()

    for b in range(NBUF):
        pltpu.async_copy(rows[b], acc_sh.at[prime_v], sem_s[b], add=True)

    def blk(blki, _):
        row0 = wid * CPT2 + blki * BLK2
        pltpu.sync_copy(src_hbm.at[pl.ds(row0, BLK2)], src_v)
        pltpu.sync_copy(dst_hbm.at[pl.ds(row0, BLK2)], dst_v)

        def quad(qi, _):
            gds = []
            for b in range(NBUF):
                pltpu.make_async_copy(rows[b], acc_sh.at[pl.ds(0, CH2)],
                                      sem_s[b]).wait()
                gds.append(pltpu.async_copy(
                    g_hbm.at[src_v.at[qi * NBUF + b]], rows[b], sem_g[b]))
            for b in range(NBUF):
                gds[b].wait()
                pltpu.async_copy(rows[b], acc_sh.at[dst_v.at[qi * NBUF + b]],
                                 sem_s[b], add=True)
            return ()

        lax.fori_loop(0, NQ, quad, ())
        return ()

    lax.fori_loop(0, CPT2 // BLK2, blk, ())
    for b in range(NBUF):
        pltpu.make_async_copy(rows[b], acc_sh.at[pl.ds(0, CH2)],
                              sem_s[b]).wait()
    plsc.subcore_barrier()
    pltpu.sync_copy(acc_sh.at[pl.ds(s * ROWS_IO, ROWS_IO)],
                    out_hbm.at[c, pl.ds(s * ROWS_IO, ROWS_IO)])


BN = 2000
NBLK = N // BN


def _tc_prep_body(deg_ref, x_ref, w_ref, g_ref, dinv_ref):
    deg = deg_ref[0] + deg_ref[1] + 1.0
    dinv = lax.rsqrt(deg)
    g_ref[...] = jnp.dot(x_ref[...], w_ref[...],
                         preferred_element_type=jnp.float32) * dinv
    dinv_ref[...] = dinv


def _tc_prep(deg2, x, w1):
    return pl.pallas_call(
        _tc_prep_body,
        grid=(NBLK,),
        in_specs=[
            pl.BlockSpec((2, BN, 1), lambda i: (0, i, 0)),
            pl.BlockSpec((BN, D), lambda i: (i, 0)),
            pl.BlockSpec((D, D), lambda i: (0, 0)),
        ],
        out_specs=[
            pl.BlockSpec((BN, D), lambda i: (i, 0)),
            pl.BlockSpec((BN, 1), lambda i: (i, 0)),
        ],
        out_shape=[jax.ShapeDtypeStruct((NP, D), jnp.float32),
                   jax.ShapeDtypeStruct((N, 1), jnp.float32)],
    )(deg2, x, w1)


def _tc_mid_body(s_ref, g1_ref, dinv_ref, b1_ref, w2_ref, g2_ref):
    dinv = dinv_ref[...]
    pre = (s_ref[0] + s_ref[1] - g1_ref[...]) * dinv + b1_ref[...]
    h = jnp.tanh(pre)
    g2_ref[...] = jnp.dot(h, w2_ref[...],
                          preferred_element_type=jnp.float32) * dinv


def _tc_mid(s, g1, dinv, b1, w2):
    return pl.pallas_call(
        _tc_mid_body,
        grid=(NBLK,),
        in_specs=[
            pl.BlockSpec((2, BN, D), lambda i: (0, i, 0)),
            pl.BlockSpec((BN, D), lambda i: (i, 0)),
            pl.BlockSpec((BN, 1), lambda i: (i, 0)),
            pl.BlockSpec((1, D), lambda i: (0, 0)),
            pl.BlockSpec((D, D), lambda i: (0, 0)),
        ],
        out_specs=pl.BlockSpec((BN, D), lambda i: (i, 0)),
        out_shape=jax.ShapeDtypeStruct((NP, D), jnp.float32),
    )(s, g1, dinv, b1, w2)


def _tc_final_body(t_ref, g2_ref, dinv_ref, b2_ref, batch_ref, wfc_ref,
                   bfc_ref, out_ref, sums_ref, cnt_ref):
    i = pl.program_id(0)
    pre = (t_ref[0] + t_ref[1] - g2_ref[...]) * dinv_ref[...] + b2_ref[...]
    h = jnp.tanh(pre)
    ids = lax.broadcasted_iota(jnp.int32, (BN, G), 1)
    ind = (ids == batch_ref[...]).astype(jnp.float32)

    @pl.when(i == 0)
    def _():
        sums_ref[...] = jnp.zeros_like(sums_ref)
        cnt_ref[...] = jnp.zeros_like(cnt_ref)

    dnums = (((0,), (0,)), ((), ()))
    sums_ref[...] += lax.dot_general(ind, h, dnums,
                                     preferred_element_type=jnp.float32)
    cnt_ref[...] += lax.dot_general(ind, jnp.ones((BN, 1), jnp.float32),
                                    dnums, preferred_element_type=jnp.float32)

    @pl.when(i == NBLK - 1)
    def _():
        pooled = sums_ref[...] / jnp.maximum(cnt_ref[...], 1.0)
        out_ref[...] = jnp.dot(pooled, wfc_ref[...],
                               preferred_element_type=jnp.float32) + bfc_ref[...]


def _tc_final(t, g2, dinv, b2, batch2d, wfc, bfc):
    return pl.pallas_call(
        _tc_final_body,
        grid=(NBLK,),
        in_specs=[
            pl.BlockSpec((2, BN, D), lambda i: (0, i, 0)),
            pl.BlockSpec((BN, D), lambda i: (i, 0)),
            pl.BlockSpec((BN, 1), lambda i: (i, 0)),
            pl.BlockSpec((1, D), lambda i: (0, 0)),
            pl.BlockSpec((BN, 1), lambda i: (i, 0)),
            pl.BlockSpec((D, O), lambda i: (0, 0)),
            pl.BlockSpec((1, O), lambda i: (0, 0)),
        ],
        out_specs=pl.BlockSpec((G, O), lambda i: (0, 0)),
        out_shape=jax.ShapeDtypeStruct((G, O), jnp.float32),
        scratch_shapes=[pltpu.VMEM((G, D), jnp.float32),
                        pltpu.VMEM((G, 1), jnp.float32)],
    )(t, g2, dinv, b2, batch2d, wfc, bfc)


def kernel(x, edge_index, batch, W1, b1, W2, b2, Wfc, bfc):
    src = edge_index[0]
    dst = edge_index[1]
    pad = E_PAD - E
    pad_idx = jnp.arange(pad, dtype=jnp.int32)
    src_p = jnp.concatenate([src, pad_idx % N])
    dst_p = jnp.concatenate([dst, N + pad_idx % (NP - N)])
    src_a = src_p.reshape(E_PAD // CH2, CH2)
    dst_a = dst_p.reshape(E_PAD // CH2, CH2)

    deg2 = _sc_degree(dst_p.reshape(E_PAD // CH, CH))
    deg2 = deg2.reshape(NC, NP)[:, :N, None]
    g1, dinv = _tc_prep(deg2, x, W1)
    s = _sc_aggregate(g1, src_a, dst_a)
    g2 = _tc_mid(s, g1, dinv, b1.reshape(1, D), W2)
    t = _sc_aggregate(g2, src_a, dst_a)
    out = _tc_final(t, g2, dinv, b2.reshape(1, D), batch.reshape(N, 1),
                    Wfc, bfc.reshape(1, O))
    return out


# speedup vs baseline: 1.0420x; 1.0194x over previous
import functools

import jax
import jax.numpy as jnp
from jax import lax
from jax.experimental import pallas as pl
from jax.experimental.pallas import tpu as pltpu
from jax.experimental.pallas import tpu_sc as plsc

N = 10000
E = 320000
D = 128
G = 64
O = 16

NC = 2
NS = 16
CH = 128

CPT = 80
E_PAD = NC * NS * CPT * CH
NP = 10240
ROWS_IO = NP // NS
ZCH = NP // NS

_MESH = plsc.VectorSubcoreMesh(core_axis_name="c", subcore_axis_name="s")


@functools.partial(
    pl.kernel,
    out_type=jax.ShapeDtypeStruct((NC * NP,), jnp.float32),
    mesh=_MESH,
    scratch_types=[
        pltpu.VMEM((CPT, CH), jnp.int32),
        pltpu.VMEM((CH,), jnp.float32),
        pltpu.VMEM((ZCH,), jnp.float32),
        pltpu.VMEM_SHARED((NP,), jnp.float32),
    ],
)
def _sc_degree(dst_hbm, out_hbm, dst_v, ones_v, zeros_v, deg_sh):
    c = lax.axis_index("c")
    s = lax.axis_index("s")
    wid = c * NS + s

    for k in range(CH // 16):
        ones_v[pl.ds(k * 16, 16)] = jnp.ones((16,), jnp.float32)

    def zbody(k, _):
        zeros_v[pl.ds(k * 16, 16)] = jnp.zeros((16,), jnp.float32)
        return ()

    lax.fori_loop(0, ZCH // 16, zbody, ())
    pltpu.sync_copy(zeros_v, deg_sh.at[pl.ds(s * ZCH, ZCH)])
    pltpu.sync_copy(dst_hbm.at[pl.ds(wid * CPT, CPT)], dst_v)
    plsc.subcore_barrier()

    def body(i, _):
        pltpu.sync_copy(ones_v, deg_sh.at[dst_v.at[i]], add=True)
        return ()

    lax.fori_loop(0, CPT, body, ())
    plsc.subcore_barrier()
    pltpu.sync_copy(deg_sh.at[pl.ds(s * ZCH, ZCH)],
                    out_hbm.at[pl.ds(c * NP + s * ZCH, ZCH)])


NBUF = 4
CH2 = 64
CPT2 = E_PAD // (NC * NS * CH2)
BLK2 = 32
NQ = BLK2 // NBUF


@functools.partial(
    pl.kernel,
    out_type=jax.ShapeDtypeStruct((NC, NP, D), jnp.float32),
    mesh=_MESH,
    scratch_types=[
        pltpu.VMEM((BLK2, CH2), jnp.int32),
        pltpu.VMEM((BLK2, CH2), jnp.int32),
        [pltpu.VMEM((CH2, D), jnp.float32) for _ in range(NBUF)],
        pltpu.VMEM_SHARED((NP, D), jnp.float32),
        [pltpu.SemaphoreType.DMA for _ in range(NBUF)],
        [pltpu.SemaphoreType.DMA for _ in range(NBUF)],
        pltpu.SemaphoreType.DMA,
    ],
)
def _sc_aggregate(g_hbm, src_hbm, dst_hbm, out_hbm,
                  src_v, dst_v, rows, acc_sh, sem_g, sem_s, sem_i):
    c = lax.axis_index("c")
    s = lax.axis_index("s")
    wid = c * NS + s

    init = pltpu.async_copy(g_hbm.at[pl.ds(s * ROWS_IO, ROWS_IO)],
                            acc_sh.at[pl.ds(s * ROWS_IO, ROWS_IO)], sem_i)
    pltpu.sync_copy(src_hbm.at[pl.ds(wid * CPT2, BLK2)], src_v)
    pltpu.sync_copy(dst_hbm.at[pl.ds(wid * CPT2, BLK2)], dst_v)
    for b in range(NBUF):
        pltpu.async_copy(g_hbm.at[src_v.at[b]], rows[b], sem_g[b])
    init.wait()
    plsc.subcore_barrier()

    def blk(blki, _):
        row0 = wid * CPT2 + blki * BLK2

        @pl.when(blki > 0)
        def _():
            pltpu.sync_copy(src_hbm.at[pl.ds(row0, BLK2)], src_v)
            pltpu.sync_copy(dst_hbm.at[pl.ds(row0, BLK2)], dst_v)

        def quad(qi, _):
            not_first = jnp.logical_or(blki > 0, qi > 0)
            gds = []
            for b in range(NBUF):
                @pl.when(not_first)
                def _():
                    pltpu.make_async_copy(rows[b], acc_sh.at[pl.ds(0, CH2)],
                                          sem_s[b]).wait()
                    pltpu.async_copy(g_hbm.at[src_v.at[qi * NBUF + b]],
                                     rows[b], sem_g[b])
                gds.append(pltpu.make_async_copy(
                    g_hbm.at[src_v.at[qi * NBUF + b]], rows[b], sem_g[b]))
            for b in range(NBUF):
                gds[b].wait()
                pltpu.async_copy(rows[b], acc_sh.at[dst_v.at[qi * NBUF + b]],
                                 sem_s[b], add=True)
            return ()

        lax.fori_loop(0, NQ, quad, ())
        return ()

    lax.fori_loop(0, CPT2 // BLK2, blk, ())
    for b in range(NBUF):
        pltpu.make_async_copy(rows[b], acc_sh.at[pl.ds(0, CH2)],
                              sem_s[b]).wait()
    plsc.subcore_barrier()
    pltpu.sync_copy(acc_sh.at[pl.ds(s * ROWS_IO, ROWS_IO)],
                    out_hbm.at[c, pl.ds(s * ROWS_IO, ROWS_IO)])


BN = 2000
NBLK = N // BN


def _tc_prep_body(deg_ref, x_ref, w_ref, g_ref, dinv_ref):
    deg = deg_ref[0] + deg_ref[1] + 1.0
    dinv = lax.rsqrt(deg)
    g_ref[...] = jnp.dot(x_ref[...], w_ref[...],
                         preferred_element_type=jnp.float32) * dinv
    dinv_ref[...] = dinv


def _tc_prep(deg2, x, w1):
    return pl.pallas_call(
        _tc_prep_body,
        grid=(NBLK,),
        in_specs=[
            pl.BlockSpec((2, BN, 1), lambda i: (0, i, 0)),
            pl.BlockSpec((BN, D), lambda i: (i, 0)),
            pl.BlockSpec((D, D), lambda i: (0, 0)),
        ],
        out_specs=[
            pl.BlockSpec((BN, D), lambda i: (i, 0)),
            pl.BlockSpec((BN, 1), lambda i: (i, 0)),
        ],
        out_shape=[jax.ShapeDtypeStruct((NP, D), jnp.float32),
                   jax.ShapeDtypeStruct((N, 1), jnp.float32)],
    )(deg2, x, w1)


def _tc_mid_body(s_ref, g1_ref, dinv_ref, b1_ref, w2_ref, g2_ref):
    dinv = dinv_ref[...]
    pre = (s_ref[0] + s_ref[1] - g1_ref[...]) * dinv + b1_ref[...]
    h = jnp.tanh(pre)
    g2_ref[...] = jnp.dot(h, w2_ref[...],
                          preferred_element_type=jnp.float32) * dinv


def _tc_mid(s, g1, dinv, b1, w2):
    return pl.pallas_call(
        _tc_mid_body,
        grid=(NBLK,),
        in_specs=[
            pl.BlockSpec((2, BN, D), lambda i: (0, i, 0)),
            pl.BlockSpec((BN, D), lambda i: (i, 0)),
            pl.BlockSpec((BN, 1), lambda i: (i, 0)),
            pl.BlockSpec((1, D), lambda i: (0, 0)),
            pl.BlockSpec((D, D), lambda i: (0, 0)),
        ],
        out_specs=pl.BlockSpec((BN, D), lambda i: (i, 0)),
        out_shape=jax.ShapeDtypeStruct((NP, D), jnp.float32),
    )(s, g1, dinv, b1, w2)


def _tc_final_body(t_ref, g2_ref, dinv_ref, b2_ref, batch_ref, wfc_ref,
                   bfc_ref, out_ref, sums_ref, cnt_ref):
    i = pl.program_id(0)
    pre = (t_ref[0] + t_ref[1] - g2_ref[...]) * dinv_ref[...] + b2_ref[...]
    h = jnp.tanh(pre)
    ids = lax.broadcasted_iota(jnp.int32, (BN, G), 1)
    ind = (ids == batch_ref[...]).astype(jnp.float32)

    @pl.when(i == 0)
    def _():
        sums_ref[...] = jnp.zeros_like(sums_ref)
        cnt_ref[...] = jnp.zeros_like(cnt_ref)

    dnums = (((0,), (0,)), ((), ()))
    sums_ref[...] += lax.dot_general(ind, h, dnums,
                                     preferred_element_type=jnp.float32)
    cnt_ref[...] += lax.dot_general(ind, jnp.ones((BN, 1), jnp.float32),
                                    dnums, preferred_element_type=jnp.float32)

    @pl.when(i == NBLK - 1)
    def _():
        pooled = sums_ref[...] / jnp.maximum(cnt_ref[...], 1.0)
        out_ref[...] = jnp.dot(pooled, wfc_ref[...],
                               preferred_element_type=jnp.float32) + bfc_ref[...]


def _tc_final(t, g2, dinv, b2, batch2d, wfc, bfc):
    return pl.pallas_call(
        _tc_final_body,
        grid=(NBLK,),
        in_specs=[
            pl.BlockSpec((2, BN, D), lambda i: (0, i, 0)),
            pl.BlockSpec((BN, D), lambda i: (i, 0)),
            pl.BlockSpec((BN, 1), lambda i: (i, 0)),
            pl.BlockSpec((1, D), lambda i: (0, 0)),
            pl.BlockSpec((BN, 1), lambda i: (i, 0)),
            pl.BlockSpec((D, O), lambda i: (0, 0)),
            pl.BlockSpec((1, O), lambda i: (0, 0)),
        ],
        out_specs=pl.BlockSpec((G, O), lambda i: (0, 0)),
        out_shape=jax.ShapeDtypeStruct((G, O), jnp.float32),
        scratch_shapes=[pltpu.VMEM((G, D), jnp.float32),
                        pltpu.VMEM((G, 1), jnp.float32)],
    )(t, g2, dinv, b2, batch2d, wfc, bfc)


def kernel(x, edge_index, batch, W1, b1, W2, b2, Wfc, bfc):
    src = edge_index[0]
    dst = edge_index[1]
    pad = E_PAD - E
    pad_idx = jnp.arange(pad, dtype=jnp.int32)
    src_p = jnp.concatenate([src, pad_idx % N])
    dst_p = jnp.concatenate([dst, N + pad_idx % (NP - N)])
    src_a = src_p.reshape(E_PAD // CH2, CH2)
    dst_a = dst_p.reshape(E_PAD // CH2, CH2)

    deg2 = _sc_degree(dst_p.reshape(E_PAD // CH, CH))
    deg2 = deg2.reshape(NC, NP)[:, :N, None]
    g1, dinv = _tc_prep(deg2, x, W1)
    s = _sc_aggregate(g1, src_a, dst_a)
    g2 = _tc_mid(s, g1, dinv, b1.reshape(1, D), W2)
    t = _sc_aggregate(g2, src_a, dst_a)
    out = _tc_final(t, g2, dinv, b2.reshape(1, D), batch.reshape(N, 1),
                    Wfc, bfc.reshape(1, O))
    return out


# speedup vs baseline: 1.0572x; 1.0146x over previous
import functools

import jax
import jax.numpy as jnp
from jax import lax
from jax.experimental import pallas as pl
from jax.experimental.pallas import tpu as pltpu
from jax.experimental.pallas import tpu_sc as plsc

N = 10000
E = 320000
D = 128
G = 64
O = 16

NC = 2
NS = 16
CH = 128

CPT = 80
E_PAD = NC * NS * CPT * CH
NP = 10240
ROWS_IO = NP // NS
ZCH = NP // NS

_MESH = plsc.VectorSubcoreMesh(core_axis_name="c", subcore_axis_name="s")


@functools.partial(
    pl.kernel,
    out_type=jax.ShapeDtypeStruct((NC * NP,), jnp.float32),
    mesh=_MESH,
    scratch_types=[
        pltpu.VMEM((CPT, CH), jnp.int32),
        pltpu.VMEM((CH,), jnp.float32),
        pltpu.VMEM((ZCH,), jnp.float32),
        pltpu.VMEM_SHARED((NP,), jnp.float32),
        pltpu.SemaphoreType.DMA,
    ],
)
def _sc_degree(dst_hbm, out_hbm, dst_v, ones_v, zeros_v, deg_sh, sem):
    c = lax.axis_index("c")
    s = lax.axis_index("s")
    wid = c * NS + s

    for k in range(CH // 16):
        ones_v[pl.ds(k * 16, 16)] = jnp.ones((16,), jnp.float32)

    def zbody(k, _):
        zeros_v[pl.ds(k * 16, 16)] = jnp.zeros((16,), jnp.float32)
        return ()

    lax.fori_loop(0, ZCH // 16, zbody, ())
    pltpu.sync_copy(zeros_v, deg_sh.at[pl.ds(s * ZCH, ZCH)])
    pltpu.sync_copy(dst_hbm.at[pl.ds(wid * CPT, CPT)], dst_v)
    plsc.subcore_barrier()

    def body(i, _):
        pltpu.async_copy(ones_v, deg_sh.at[dst_v.at[i]], sem, add=True)
        return ()

    lax.fori_loop(0, CPT, body, ())

    def drain(i, _):
        pltpu.make_async_copy(ones_v, deg_sh.at[pl.ds(0, CH)], sem).wait()
        return ()

    lax.fori_loop(0, CPT, drain, ())
    plsc.subcore_barrier()
    pltpu.sync_copy(deg_sh.at[pl.ds(s * ZCH, ZCH)],
                    out_hbm.at[pl.ds(c * NP + s * ZCH, ZCH)])


NBUF = 4
CH2 = 64
CPT2 = E_PAD // (NC * NS * CH2)
BLK2 = 32
NQ = BLK2 // NBUF


@functools.partial(
    pl.kernel,
    out_type=jax.ShapeDtypeStruct((NC, NP, D), jnp.float32),
    mesh=_MESH,
    scratch_types=[
        pltpu.VMEM((BLK2, CH2), jnp.int32),
        pltpu.VMEM((BLK2, CH2), jnp.int32),
        [pltpu.VMEM((CH2, D), jnp.float32) for _ in range(NBUF)],
        pltpu.VMEM_SHARED((NP, D), jnp.float32),
        [pltpu.SemaphoreType.DMA for _ in range(NBUF)],
        [pltpu.SemaphoreType.DMA for _ in range(NBUF)],
        pltpu.SemaphoreType.DMA,
    ],
)
def _sc_aggregate(g_hbm, src_hbm, dst_hbm, out_hbm,
                  src_v, dst_v, rows, acc_sh, sem_g, sem_s, sem_i):
    c = lax.axis_index("c")
    s = lax.axis_index("s")
    wid = c * NS + s

    init = pltpu.async_copy(g_hbm.at[pl.ds(s * ROWS_IO, ROWS_IO)],
                            acc_sh.at[pl.ds(s * ROWS_IO, ROWS_IO)], sem_i)
    pltpu.sync_copy(src_hbm.at[pl.ds(wid * CPT2, BLK2)], src_v)
    pltpu.sync_copy(dst_hbm.at[pl.ds(wid * CPT2, BLK2)], dst_v)
    for b in range(NBUF):
        pltpu.async_copy(g_hbm.at[src_v.at[b]], rows[b], sem_g[b])
    init.wait()
    plsc.subcore_barrier()

    def blk(blki, _):
        row0 = wid * CPT2 + blki * BLK2

        @pl.when(blki > 0)
        def _():
            pltpu.sync_copy(src_hbm.at[pl.ds(row0, BLK2)], src_v)
            pltpu.sync_copy(dst_hbm.at[pl.ds(row0, BLK2)], dst_v)

        def quad(qi, _):
            not_first = jnp.logical_or(blki > 0, qi > 0)
            gds = []
            for b in range(NBUF):
                @pl.when(not_first)
                def _():
                    pltpu.make_async_copy(rows[b], acc_sh.at[pl.ds(0, CH2)],
                                          sem_s[b]).wait()
                    pltpu.async_copy(g_hbm.at[src_v.at[qi * NBUF + b]],
                                     rows[b], sem_g[b])
                gds.append(pltpu.make_async_copy(
                    g_hbm.at[src_v.at[qi * NBUF + b]], rows[b], sem_g[b]))
            for b in range(NBUF):
                gds[b].wait()
                pltpu.async_copy(rows[b], acc_sh.at[dst_v.at[qi * NBUF + b]],
                                 sem_s[b], add=True)
            return ()

        lax.fori_loop(0, NQ, quad, ())
        return ()

    lax.fori_loop(0, CPT2 // BLK2, blk, ())
    for b in range(NBUF):
        pltpu.make_async_copy(rows[b], acc_sh.at[pl.ds(0, CH2)],
                              sem_s[b]).wait()
    plsc.subcore_barrier()
    pltpu.sync_copy(acc_sh.at[pl.ds(s * ROWS_IO, ROWS_IO)],
                    out_hbm.at[c, pl.ds(s * ROWS_IO, ROWS_IO)])


BN = 2000
NBLK = N // BN


def _tc_prep_body(deg_ref, x_ref, w_ref, g_ref, dinv_ref):
    deg = deg_ref[0] + deg_ref[1] + 1.0
    dinv = lax.rsqrt(deg)
    g_ref[...] = jnp.dot(x_ref[...], w_ref[...],
                         preferred_element_type=jnp.float32) * dinv
    dinv_ref[...] = dinv


def _tc_prep(deg2, x, w1):
    return pl.pallas_call(
        _tc_prep_body,
        grid=(NBLK,),
        in_specs=[
            pl.BlockSpec((2, BN, 1), lambda i: (0, i, 0)),
            pl.BlockSpec((BN, D), lambda i: (i, 0)),
            pl.BlockSpec((D, D), lambda i: (0, 0)),
        ],
        out_specs=[
            pl.BlockSpec((BN, D), lambda i: (i, 0)),
            pl.BlockSpec((BN, 1), lambda i: (i, 0)),
        ],
        out_shape=[jax.ShapeDtypeStruct((NP, D), jnp.float32),
                   jax.ShapeDtypeStruct((N, 1), jnp.float32)],
    )(deg2, x, w1)


def _tc_mid_body(s_ref, g1_ref, dinv_ref, b1_ref, w2_ref, g2_ref):
    dinv = dinv_ref[...]
    pre = (s_ref[0] + s_ref[1] - g1_ref[...]) * dinv + b1_ref[...]
    h = jnp.tanh(pre)
    g2_ref[...] = jnp.dot(h, w2_ref[...],
                          preferred_element_type=jnp.float32) * dinv


def _tc_mid(s, g1, dinv, b1, w2):
    return pl.pallas_call(
        _tc_mid_body,
        grid=(NBLK,),
        in_specs=[
            pl.BlockSpec((2, BN, D), lambda i: (0, i, 0)),
            pl.BlockSpec((BN, D), lambda i: (i, 0)),
            pl.BlockSpec((BN, 1), lambda i: (i, 0)),
            pl.BlockSpec((1, D), lambda i: (0, 0)),
            pl.BlockSpec((D, D), lambda i: (0, 0)),
        ],
        out_specs=pl.BlockSpec((BN, D), lambda i: (i, 0)),
        out_shape=jax.ShapeDtypeStruct((NP, D), jnp.float32),
    )(s, g1, dinv, b1, w2)


def _tc_final_body(t_ref, g2_ref, dinv_ref, b2_ref, batch_ref, wfc_ref,
                   bfc_ref, out_ref, sums_ref, cnt_ref):
    i = pl.program_id(0)
    pre = (t_ref[0] + t_ref[1] - g2_ref[...]) * dinv_ref[...] + b2_ref[...]
    h = jnp.tanh(pre)
    ids = lax.broadcasted_iota(jnp.int32, (BN, G), 1)
    ind = (ids == batch_ref[...]).astype(jnp.float32)

    @pl.when(i == 0)
    def _():
        sums_ref[...] = jnp.zeros_like(sums_ref)
        cnt_ref[...] = jnp.zeros_like(cnt_ref)

    dnums = (((0,), (0,)), ((), ()))
    sums_ref[...] += lax.dot_general(ind, h, dnums,
                                     preferred_element_type=jnp.float32)
    cnt_ref[...] += lax.dot_general(ind, jnp.ones((BN, 1), jnp.float32),
                                    dnums, preferred_element_type=jnp.float32)

    @pl.when(i == NBLK - 1)
    def _():
        pooled = sums_ref[...] / jnp.maximum(cnt_ref[...], 1.0)
        out_ref[...] = jnp.dot(pooled, wfc_ref[...],
                               preferred_element_type=jnp.float32) + bfc_ref[...]


def _tc_final(t, g2, dinv, b2, batch2d, wfc, bfc):
    return pl.pallas_call(
        _tc_final_body,
        grid=(NBLK,),
        in_specs=[
            pl.BlockSpec((2, BN, D), lambda i: (0, i, 0)),
            pl.BlockSpec((BN, D), lambda i: (i, 0)),
            pl.BlockSpec((BN, 1), lambda i: (i, 0)),
            pl.BlockSpec((1, D), lambda i: (0, 0)),
            pl.BlockSpec((BN, 1), lambda i: (i, 0)),
            pl.BlockSpec((D, O), lambda i: (0, 0)),
            pl.BlockSpec((1, O), lambda i: (0, 0)),
        ],
        out_specs=pl.BlockSpec((G, O), lambda i: (0, 0)),
        out_shape=jax.ShapeDtypeStruct((G, O), jnp.float32),
        scratch_shapes=[pltpu.VMEM((G, D), jnp.float32),
                        pltpu.VMEM((G, 1), jnp.float32)],
    )(t, g2, dinv, b2, batch2d, wfc, bfc)


def kernel(x, edge_index, batch, W1, b1, W2, b2, Wfc, bfc):
    src = edge_index[0]
    dst = edge_index[1]
    pad = E_PAD - E
    pad_idx = jnp.arange(pad, dtype=jnp.int32)
    src_p = jnp.concatenate([src, pad_idx % N])
    dst_p = jnp.concatenate([dst, N + pad_idx % (NP - N)])
    src_a = src_p.reshape(E_PAD // CH2, CH2)
    dst_a = dst_p.reshape(E_PAD // CH2, CH2)

    deg2 = _sc_degree(dst_p.reshape(E_PAD // CH, CH))
    deg2 = deg2.reshape(NC, NP)[:, :N, None]
    g1, dinv = _tc_prep(deg2, x, W1)
    s = _sc_aggregate(g1, src_a, dst_a)
    g2 = _tc_mid(s, g1, dinv, b1.reshape(1, D), W2)
    t = _sc_aggregate(g2, src_a, dst_a)
    out = _tc_final(t, g2, dinv, b2.reshape(1, D), batch.reshape(N, 1),
                    Wfc, bfc.reshape(1, O))
    return out
